# baseline (speedup 1.0000x reference)
import functools

import jax
import jax.numpy as jnp
from jax import lax
from jax.experimental import pallas as pl
from jax.experimental.pallas import tpu as pltpu
from jax.experimental.pallas import tpu_sc as plsc

N = 10000
D = 128
E = 320000

NC = 2
NS = 16
NW = NC * NS

CHUNK = 128
CPT = 80
HCPT = CPT // 2
HPAIR = HCPT // 2
NCHUNKS_P = NW * CPT
NCHUNKS = E // CHUNK
RPT = 72
EXTRA_BASE = NW * RPT
EPW = CPT - RPT
NEXTRA = NW * EPW
NPADC = NCHUNKS_P - NCHUNKS
NDUMP = 512
NACC = N + NDUMP

ROWS_A = 624
ROWS_REM = N - NS * ROWS_A
ZTAIL = ROWS_A - 4 * CHUNK


def _sc_partial_segment_sum(x, ei3, ex3):
    mesh = plsc.VectorSubcoreMesh(core_axis_name="c", subcore_axis_name="s")

    @functools.partial(
        pl.kernel,
        out_type=jax.ShapeDtypeStruct((NC, N, D), jnp.float32),
        mesh=mesh,
        scratch_types=[
            pltpu.VMEM_SHARED((NACC, D), jnp.float32),
            pltpu.VMEM((HCPT, CHUNK), jnp.int32),
            pltpu.VMEM((HCPT, CHUNK), jnp.int32),
            pltpu.VMEM((CHUNK, D), jnp.float32),
            pltpu.VMEM((CHUNK, D), jnp.float32),
            pltpu.SemaphoreType.DMA,
            pltpu.SemaphoreType.DMA,
            pltpu.SemaphoreType.DMA,
            pltpu.SemaphoreType.DMA,
            pltpu.SemaphoreType.DMA,
            pltpu.SemaphoreType.DMA,
            pltpu.SemaphoreType.DMA,
            pltpu.SemaphoreType.DMA,
            pltpu.SemaphoreType.DMA,
            pltpu.SemaphoreType.DMA,
        ],
    )
    def k(x_hbm, ei_hbm, ex_hbm, out_hbm, acc, srcb, dstb, rows0, rows1,
          gsem0, gsem1, ssem0, ssem1, qsem0, qsem1, qsem2, qsem3,
          zsem0, zsem1):
        c = lax.axis_index("c")
        s = lax.axis_index("s")
        w = c * NS + s

        def idx_load_h0(sem_pair_start):
            cps = [
                pltpu.make_async_copy(ei_hbm.at[0, pl.ds(w * RPT, HCPT)],
                                      srcb, gsem0),
                pltpu.make_async_copy(ei_hbm.at[1, pl.ds(w * RPT, HCPT)],
                                      dstb, gsem1),
            ]
            for cp in cps:
                cp.start() if sem_pair_start else cp.wait()

        idx_load_h0(True)

        zeros16 = jnp.zeros((16,), jnp.float32)

        @pl.loop(0, CHUNK)
        def _(r):
            @pl.loop(0, D, step=16)
            def _(cc):
                rows0[r, pl.ds(cc, 16)] = zeros16

        def zero_copies(start):
            cps = [
                pltpu.make_async_copy(
                    rows0, acc.at[pl.ds(s * ROWS_A + r0, CHUNK)], ssem0)
                for r0 in range(0, 4 * CHUNK, CHUNK)
            ]
            cps.append(pltpu.make_async_copy(
                rows0.at[pl.ds(0, ZTAIL)],
                acc.at[pl.ds(s * ROWS_A + 4 * CHUNK, ZTAIL)], ssem1))
            for cp in cps:
                cp.start() if start else cp.wait()

        zero_copies(True)
        zero_copies(False)

        @pl.when(s == 0)
        def _():
            pltpu.sync_copy(rows0.at[pl.ds(0, ROWS_REM)],
                            acc.at[pl.ds(NS * ROWS_A, ROWS_REM)])

        plsc.subcore_barrier()
        idx_load_h0(False)

        def gather(t, rbuf, sem):
            return pltpu.make_async_copy(x_hbm.at[srcb.at[t]], rbuf, sem)

        def scatter(t, rbuf, sem):
            return pltpu.make_async_copy(rbuf, acc.at[dstb.at[t]], sem)

        QR = CHUNK // 4

        def gather_q(t, q, rbuf, sem):
            return pltpu.make_async_copy(
                x_hbm.at[srcb.at[t].at[pl.ds(q * QR, QR)]],
                rbuf.at[pl.ds(q * QR, QR)], sem)

        sems0 = (gsem0, gsem1, ssem0, ssem1)
        sems1 = (qsem0, qsem1, qsem2, qsem3)

        def pipeline_half():
            for q in range(4):
                gather_q(0, q, rows0, sems0[q]).start()
                gather_q(1, q, rows1, sems1[q]).start()

            @pl.loop(0, HPAIR)
            def _(p):
                t0 = 2 * p
                t1 = t0 + 1
                for q in range(4):
                    gather_q(t0, q, rows0, sems0[q]).wait()
                scatter(t0, rows0, zsem0).start(add=True)
                for q in range(4):
                    gather_q(t1, q, rows1, sems1[q]).wait()
                scatter(t1, rows1, zsem1).start(add=True)
                scatter(t0, rows0, zsem0).wait()

                @pl.when(p < HPAIR - 1)
                def _():
                    for q in range(4):
                        gather_q(t0 + 2, q, rows0, sems0[q]).start()

                scatter(t1, rows1, zsem1).wait()

                @pl.when(p < HPAIR - 1)
                def _():
                    for q in range(4):
                        gather_q(t1 + 2, q, rows1, sems1[q]).start()

        pipeline_half()

        r2 = RPT - HCPT

        def idx_load_h1(start):
            cps = [
                pltpu.make_async_copy(ei_hbm.at[0, pl.ds(w * RPT + HCPT, r2)],
                                      srcb.at[pl.ds(0, r2)], gsem0),
                pltpu.make_async_copy(ei_hbm.at[1, pl.ds(w * RPT + HCPT, r2)],
                                      dstb.at[pl.ds(0, r2)], gsem1),
                pltpu.make_async_copy(ex_hbm.at[0, pl.ds(w * EPW, EPW)],
                                      srcb.at[pl.ds(r2, EPW)], ssem0),
                pltpu.make_async_copy(ex_hbm.at[1, pl.ds(w * EPW, EPW)],
                                      dstb.at[pl.ds(r2, EPW)], ssem1),
            ]
            for cp in cps:
                cp.start() if start else cp.wait()

        idx_load_h1(True)
        idx_load_h1(False)

        pipeline_half()

        plsc.subcore_barrier()

        pltpu.sync_copy(
            acc.at[pl.ds(s * ROWS_A, ROWS_A)],
            out_hbm.at[c, pl.ds(s * ROWS_A, ROWS_A)],
        )

        @pl.when(s == 0)
        def _():
            pltpu.sync_copy(
                acc.at[pl.ds(NS * ROWS_A, ROWS_REM)],
                out_hbm.at[c, pl.ds(NS * ROWS_A, ROWS_REM)],
            )

    return k(x, ei3, ex3)


def _tc_mlp_body(eps_ref, x_ref, a0_ref, a1_ref, w1_ref, b1_ref, w2_ref,
                 b2_ref, o_ref, *, relu_out):
    h = (1.0 + eps_ref[0, 0]) * x_ref[...] + a0_ref[...] + a1_ref[...]
    h = jnp.dot(h, w1_ref[...], preferred_element_type=jnp.float32)
    h = jnp.maximum(h + b1_ref[...], 0.0)
    o = jnp.dot(h, w2_ref[...], preferred_element_type=jnp.float32)
    o = o + b2_ref[...]
    if relu_out:
        o = jnp.maximum(o, 0.0)
    o_ref[...] = o


def _tc_gin_mlp(x, agg_partials, w1, b1, w2, b2, eps, relu_out):
    blk = 2000
    body = functools.partial(_tc_mlp_body, relu_out=relu_out)
    return pl.pallas_call(
        body,
        grid=(N // blk,),
        in_specs=[
            pl.BlockSpec((1, 1), lambda i: (0, 0), memory_space=pltpu.SMEM),
            pl.BlockSpec((blk, D), lambda i: (i, 0)),
            pl.BlockSpec((blk, D), lambda i: (i, 0)),
            pl.BlockSpec((blk, D), lambda i: (i, 0)),
            pl.BlockSpec((D, D), lambda i: (0, 0)),
            pl.BlockSpec((1, D), lambda i: (0, 0)),
            pl.BlockSpec((D, D), lambda i: (0, 0)),
            pl.BlockSpec((1, D), lambda i: (0, 0)),
        ],
        out_specs=pl.BlockSpec((blk, D), lambda i: (i, 0)),
        out_shape=jax.ShapeDtypeStruct((N, D), jnp.float32),
    )(
        eps.reshape(1, 1), x, agg_partials[0], agg_partials[1],
        w1, b1.reshape(1, D), w2, b2.reshape(1, D),
    )


def kernel(features, edge_index, W1a, b1a, W2a, b2a, eps1,
           W1b, b1b, W2b, b2b, eps2):
    real3 = edge_index.reshape(2, NCHUNKS, CHUNK)
    i = jnp.arange(NPADC * CHUNK, dtype=jnp.int32)
    pads = jnp.stack([(i * 37) % N, N + (i * 7) % NDUMP]).reshape(2, NPADC,
                                                                  CHUNK)
    ex3 = jnp.concatenate([real3[:, EXTRA_BASE:], pads], axis=1)

    agg1 = _sc_partial_segment_sum(features, real3, ex3)
    x1 = _tc_gin_mlp(features, agg1, W1a, b1a, W2a, b2a, eps1, relu_out=True)
    agg2 = _sc_partial_segment_sum(x1, real3, ex3)
    return _tc_gin_mlp(x1, agg2, W1b, b1b, W2b, b2b, eps2, relu_out=False)


# speedup vs baseline: 1.0204x; 1.0204x over previous
import functools

import jax
import jax.numpy as jnp
from jax import lax
from jax.experimental import pallas as pl
from jax.experimental.pallas import tpu as pltpu
from jax.experimental.pallas import tpu_sc as plsc

N = 10000
D = 128
E = 320000

NC = 2
NS = 16
NW = NC * NS

CHUNK = 128
CPT = 80
HCPT = CPT // 2
HPAIR = HCPT // 2
NCHUNKS_P = NW * CPT
NCHUNKS = E // CHUNK
RPT = 72
EXTRA_BASE = NW * RPT
EPW = CPT - RPT
NEXTRA = NW * EPW
NPADC = NCHUNKS_P - NCHUNKS
NDUMP = 512
NACC = N + NDUMP

ROWS_A = 624
ROWS_REM = N - NS * ROWS_A
ZTAIL = ROWS_A - 4 * CHUNK


def _sc_partial_segment_sum(x, ei3, ex3):
    mesh = plsc.VectorSubcoreMesh(core_axis_name="c", subcore_axis_name="s")

    @functools.partial(
        pl.kernel,
        out_type=jax.ShapeDtypeStruct((NC, N, D), jnp.float32),
        mesh=mesh,
        scratch_types=[
            pltpu.VMEM_SHARED((NACC, D), jnp.float32),
            pltpu.VMEM((HCPT, CHUNK), jnp.int32),
            pltpu.VMEM((HCPT, CHUNK), jnp.int32),
            pltpu.VMEM((CHUNK, D), jnp.float32),
            pltpu.VMEM((CHUNK, D), jnp.float32),
            pltpu.SemaphoreType.DMA,
            pltpu.SemaphoreType.DMA,
            pltpu.SemaphoreType.DMA,
            pltpu.SemaphoreType.DMA,
            pltpu.SemaphoreType.DMA,
            pltpu.SemaphoreType.DMA,
            pltpu.SemaphoreType.DMA,
            pltpu.SemaphoreType.DMA,
            pltpu.SemaphoreType.DMA,
            pltpu.SemaphoreType.DMA,
        ],
    )
    def k(x_hbm, ei_hbm, ex_hbm, out_hbm, acc, srcb, dstb, rows0, rows1,
          gsem0, gsem1, ssem0, ssem1, qsem0, qsem1, qsem2, qsem3,
          zsem0, zsem1):
        c = lax.axis_index("c")
        s = lax.axis_index("s")
        w = c * NS + s

        def idx_load_h0(sem_pair_start):
            cps = [
                pltpu.make_async_copy(ei_hbm.at[0, pl.ds(w * RPT, HCPT)],
                                      srcb, gsem0),
                pltpu.make_async_copy(ei_hbm.at[1, pl.ds(w * RPT, HCPT)],
                                      dstb, gsem1),
            ]
            for cp in cps:
                cp.start() if sem_pair_start else cp.wait()

        idx_load_h0(True)

        zeros16 = jnp.zeros((16,), jnp.float32)

        @pl.loop(0, CHUNK)
        def _(r):
            @pl.loop(0, D, step=16)
            def _(cc):
                rows0[r, pl.ds(cc, 16)] = zeros16

        def zero_copies(start):
            cps = [
                pltpu.make_async_copy(
                    rows0, acc.at[pl.ds(s * ROWS_A + r0, CHUNK)], ssem0)
                for r0 in range(0, 4 * CHUNK, CHUNK)
            ]
            cps.append(pltpu.make_async_copy(
                rows0.at[pl.ds(0, ZTAIL)],
                acc.at[pl.ds(s * ROWS_A + 4 * CHUNK, ZTAIL)], ssem1))
            for cp in cps:
                cp.start() if start else cp.wait()

        zero_copies(True)
        zero_copies(False)

        @pl.when(s == 0)
        def _():
            pltpu.sync_copy(rows0.at[pl.ds(0, ROWS_REM)],
                            acc.at[pl.ds(NS * ROWS_A, ROWS_REM)])

        plsc.subcore_barrier()
        idx_load_h0(False)

        def gather(t, rbuf, sem):
            return pltpu.make_async_copy(x_hbm.at[srcb.at[t]], rbuf, sem)

        def scatter(t, rbuf, sem):
            return pltpu.make_async_copy(rbuf, acc.at[dstb.at[t]], sem)

        def pipeline_half():
            gather(0, rows0, gsem0).start()

            @pl.loop(0, HPAIR)
            def _(p):
                t0 = 2 * p
                t1 = t0 + 1
                gather(t0, rows0, gsem0).wait()
                gather(t1, rows1, gsem1).start()
                scatter(t0, rows0, zsem0).start(add=True)
                gather(t1, rows1, gsem1).wait()
                scatter(t1, rows1, zsem1).start(add=True)
                scatter(t0, rows0, zsem0).wait()

                @pl.when(p < HPAIR - 1)
                def _():
                    gather(t0 + 2, rows0, gsem0).start()

                scatter(t1, rows1, zsem1).wait()

        pipeline_half()

        r2 = RPT - HCPT

        def idx_load_h1(start):
            cps = [
                pltpu.make_async_copy(ei_hbm.at[0, pl.ds(w * RPT + HCPT, r2)],
                                      srcb.at[pl.ds(0, r2)], gsem0),
                pltpu.make_async_copy(ei_hbm.at[1, pl.ds(w * RPT + HCPT, r2)],
                                      dstb.at[pl.ds(0, r2)], gsem1),
                pltpu.make_async_copy(ex_hbm.at[0, pl.ds(w * EPW, EPW)],
                                      srcb.at[pl.ds(r2, EPW)], ssem0),
                pltpu.make_async_copy(ex_hbm.at[1, pl.ds(w * EPW, EPW)],
                                      dstb.at[pl.ds(r2, EPW)], ssem1),
            ]
            for cp in cps:
                cp.start() if start else cp.wait()

        idx_load_h1(True)
        idx_load_h1(False)

        pipeline_half()

        plsc.subcore_barrier()

        pltpu.sync_copy(
            acc.at[pl.ds(s * ROWS_A, ROWS_A)],
            out_hbm.at[c, pl.ds(s * ROWS_A, ROWS_A)],
        )

        @pl.when(s == 0)
        def _():
            pltpu.sync_copy(
                acc.at[pl.ds(NS * ROWS_A, ROWS_REM)],
                out_hbm.at[c, pl.ds(NS * ROWS_A, ROWS_REM)],
            )

    return k(x, ei3, ex3)


def _tc_mlp_body(eps_ref, x_ref, a0_ref, a1_ref, w1_ref, b1_ref, w2_ref,
                 b2_ref, o_ref, *, relu_out):
    h = (1.0 + eps_ref[0, 0]) * x_ref[...] + a0_ref[...] + a1_ref[...]
    h = jnp.dot(h, w1_ref[...], preferred_element_type=jnp.float32)
    h = jnp.maximum(h + b1_ref[...], 0.0)
    o = jnp.dot(h, w2_ref[...], preferred_element_type=jnp.float32)
    o = o + b2_ref[...]
    if relu_out:
        o = jnp.maximum(o, 0.0)
    o_ref[...] = o


def _tc_gin_mlp(x, agg_partials, w1, b1, w2, b2, eps, relu_out):
    blk = 2000
    body = functools.partial(_tc_mlp_body, relu_out=relu_out)
    return pl.pallas_call(
        body,
        grid=(N // blk,),
        in_specs=[
            pl.BlockSpec((1, 1), lambda i: (0, 0), memory_space=pltpu.SMEM),
            pl.BlockSpec((blk, D), lambda i: (i, 0)),
            pl.BlockSpec((blk, D), lambda i: (i, 0)),
            pl.BlockSpec((blk, D), lambda i: (i, 0)),
            pl.BlockSpec((D, D), lambda i: (0, 0)),
            pl.BlockSpec((1, D), lambda i: (0, 0)),
            pl.BlockSpec((D, D), lambda i: (0, 0)),
            pl.BlockSpec((1, D), lambda i: (0, 0)),
        ],
        out_specs=pl.BlockSpec((blk, D), lambda i: (i, 0)),
        out_shape=jax.ShapeDtypeStruct((N, D), jnp.float32),
    )(
        eps.reshape(1, 1), x, agg_partials[0], agg_partials[1],
        w1, b1.reshape(1, D), w2, b2.reshape(1, D),
    )


def kernel(features, edge_index, W1a, b1a, W2a, b2a, eps1,
           W1b, b1b, W2b, b2b, eps2):
    real3 = edge_index.reshape(2, NCHUNKS, CHUNK)
    i = jnp.arange(NPADC * CHUNK, dtype=jnp.int32)
    pads = jnp.stack([(i * 37) % N, N + (i * 7) % NDUMP]).reshape(2, NPADC,
                                                                  CHUNK)
    ex3 = jnp.concatenate([real3[:, EXTRA_BASE:], pads], axis=1)

    agg1 = _sc_partial_segment_sum(features, real3, ex3)
    x1 = _tc_gin_mlp(features, agg1, W1a, b1a, W2a, b2a, eps1, relu_out=True)
    agg2 = _sc_partial_segment_sum(x1, real3, ex3)
    return _tc_gin_mlp(x1, agg2, W1b, b1b, W2b, b2b, eps2, relu_out=False)


# speedup vs baseline: 1.0229x; 1.0024x over previous
import functools

import jax
import jax.numpy as jnp
from jax import lax
from jax.experimental import pallas as pl
from jax.experimental.pallas import tpu as pltpu
from jax.experimental.pallas import tpu_sc as plsc

N = 10000
D = 128
E = 320000

NC = 2
NS = 16
NW = NC * NS

CHUNK = 128
CPT = 80
HCPT = CPT // 2
HPAIR = HCPT // 2
NCHUNKS_P = NW * CPT
NCHUNKS = E // CHUNK
RPT = 72
EXTRA_BASE = NW * RPT
EPW = CPT - RPT
NEXTRA = NW * EPW
NPADC = NCHUNKS_P - NCHUNKS
NDUMP = 512
NACC = N + NDUMP

ROWS_A = 624
ROWS_REM = N - NS * ROWS_A
ZTAIL = ROWS_A - 4 * CHUNK


def _sc_partial_segment_sum(x, ei3, ex3):
    mesh = plsc.VectorSubcoreMesh(core_axis_name="c", subcore_axis_name="s")

    @functools.partial(
        pl.kernel,
        out_type=jax.ShapeDtypeStruct((NC, N, D), jnp.float32),
        mesh=mesh,
        scratch_types=[
            pltpu.VMEM_SHARED((NACC, D), jnp.float32),
            pltpu.VMEM((HCPT, CHUNK), jnp.int32),
            pltpu.VMEM((HCPT, CHUNK), jnp.int32),
            pltpu.VMEM((CHUNK, D), jnp.float32),
            pltpu.VMEM((CHUNK, D), jnp.float32),
            pltpu.SemaphoreType.DMA,
            pltpu.SemaphoreType.DMA,
            pltpu.SemaphoreType.DMA,
            pltpu.SemaphoreType.DMA,
            pltpu.SemaphoreType.DMA,
            pltpu.SemaphoreType.DMA,
        ],
    )
    def k(x_hbm, ei_hbm, ex_hbm, out_hbm, acc, srcb, dstb, rows0, rows1,
          gsem0, gsem1, ssem0, ssem1, zsem0, zsem1):
        c = lax.axis_index("c")
        s = lax.axis_index("s")
        w = c * NS + s

        def idx_load_h0(sem_pair_start):
            cps = [
                pltpu.make_async_copy(ei_hbm.at[0, pl.ds(w * RPT, HCPT)],
                                      srcb, gsem0),
                pltpu.make_async_copy(ei_hbm.at[1, pl.ds(w * RPT, HCPT)],
                                      dstb, gsem1),
            ]
            for cp in cps:
                cp.start() if sem_pair_start else cp.wait()

        idx_load_h0(True)

        zeros16 = jnp.zeros((16,), jnp.float32)

        @pl.loop(0, CHUNK)
        def _(r):
            @pl.loop(0, D, step=16)
            def _(cc):
                rows0[r, pl.ds(cc, 16)] = zeros16

        def zero_copies(start):
            cps = [
                pltpu.make_async_copy(
                    rows0, acc.at[pl.ds(s * ROWS_A + r0, CHUNK)], ssem0)
                for r0 in range(0, 4 * CHUNK, CHUNK)
            ]
            cps.append(pltpu.make_async_copy(
                rows0.at[pl.ds(0, ZTAIL)],
                acc.at[pl.ds(s * ROWS_A + 4 * CHUNK, ZTAIL)], ssem1))
            for cp in cps:
                cp.start() if start else cp.wait()

        zero_copies(True)
        zero_copies(False)

        @pl.when(s == 0)
        def _():
            pltpu.sync_copy(rows0.at[pl.ds(0, ROWS_REM)],
                            acc.at[pl.ds(NS * ROWS_A, ROWS_REM)])

        plsc.subcore_barrier()
        idx_load_h0(False)

        def gather(t, rbuf, sem):
            return pltpu.make_async_copy(x_hbm.at[srcb.at[t]], rbuf, sem)

        def scatter(t, rbuf, sem):
            return pltpu.make_async_copy(rbuf, acc.at[dstb.at[t]], sem)

        def pipeline_half():
            gather(0, rows0, gsem0).start()

            @pl.loop(0, HPAIR)
            def _(p):
                t0 = 2 * p
                t1 = t0 + 1
                gather(t0, rows0, gsem0).wait()
                gather(t1, rows1, gsem1).start()
                scatter(t0, rows0, zsem0).start(add=True)
                gather(t1, rows1, gsem1).wait()
                scatter(t1, rows1, zsem1).start(add=True)
                scatter(t0, rows0, zsem0).wait()

                @pl.when(p < HPAIR - 1)
                def _():
                    gather(t0 + 2, rows0, gsem0).start()

                scatter(t1, rows1, zsem1).wait()

        pipeline_half()

        r2 = RPT - HCPT

        def idx_load_h1(start):
            cps = [
                pltpu.make_async_copy(ei_hbm.at[0, pl.ds(w * RPT + HCPT, r2)],
                                      srcb.at[pl.ds(0, r2)], gsem0),
                pltpu.make_async_copy(ei_hbm.at[1, pl.ds(w * RPT + HCPT, r2)],
                                      dstb.at[pl.ds(0, r2)], gsem1),
                pltpu.make_async_copy(ex_hbm.at[0, pl.ds(w * EPW, EPW)],
                                      srcb.at[pl.ds(r2, EPW)], ssem0),
                pltpu.make_async_copy(ex_hbm.at[1, pl.ds(w * EPW, EPW)],
                                      dstb.at[pl.ds(r2, EPW)], ssem1),
            ]
            for cp in cps:
                cp.start() if start else cp.wait()

        idx_load_h1(True)
        idx_load_h1(False)

        pipeline_half()

        plsc.subcore_barrier()

        pltpu.sync_copy(
            acc.at[pl.ds(s * ROWS_A, ROWS_A)],
            out_hbm.at[c, pl.ds(s * ROWS_A, ROWS_A)],
        )

        @pl.when(s == 0)
        def _():
            pltpu.sync_copy(
                acc.at[pl.ds(NS * ROWS_A, ROWS_REM)],
                out_hbm.at[c, pl.ds(NS * ROWS_A, ROWS_REM)],
            )

    return k(x, ei3, ex3)


def _tc_mlp_body(eps_ref, x_ref, a0_ref, a1_ref, w1_ref, b1_ref, w2_ref,
                 b2_ref, o_ref, *, relu_out):
    h = (1.0 + eps_ref[0, 0]) * x_ref[...] + a0_ref[...] + a1_ref[...]
    h = jnp.dot(h, w1_ref[...], preferred_element_type=jnp.float32)
    h = jnp.maximum(h + b1_ref[...], 0.0)
    o = jnp.dot(h, w2_ref[...], preferred_element_type=jnp.float32)
    o = o + b2_ref[...]
    if relu_out:
        o = jnp.maximum(o, 0.0)
    o_ref[...] = o


def _tc_gin_mlp(x, agg_partials, w1, b1, w2, b2, eps, relu_out):
    blk = 2000
    body = functools.partial(_tc_mlp_body, relu_out=relu_out)
    return pl.pallas_call(
        body,
        grid=(N // blk,),
        in_specs=[
            pl.BlockSpec((1, 1), lambda i: (0, 0), memory_space=pltpu.SMEM),
            pl.BlockSpec((blk, D), lambda i: (i, 0)),
            pl.BlockSpec((blk, D), lambda i: (i, 0)),
            pl.BlockSpec((blk, D), lambda i: (i, 0)),
            pl.BlockSpec((D, D), lambda i: (0, 0)),
            pl.BlockSpec((1, D), lambda i: (0, 0)),
            pl.BlockSpec((D, D), lambda i: (0, 0)),
            pl.BlockSpec((1, D), lambda i: (0, 0)),
        ],
        out_specs=pl.BlockSpec((blk, D), lambda i: (i, 0)),
        out_shape=jax.ShapeDtypeStruct((N, D), jnp.float32),
    )(
        eps.reshape(1, 1), x, agg_partials[0], agg_partials[1],
        w1, b1.reshape(1, D), w2, b2.reshape(1, D),
    )


def kernel(features, edge_index, W1a, b1a, W2a, b2a, eps1,
           W1b, b1b, W2b, b2b, eps2):
    real3 = edge_index.reshape(2, NCHUNKS, CHUNK)
    i = jnp.arange(NPADC * CHUNK, dtype=jnp.int32)
    pads = jnp.stack([(i * 37) % N, N + (i * 7) % NDUMP]).reshape(2, NPADC,
                                                                  CHUNK)
    ex3 = jnp.concatenate([real3[:, EXTRA_BASE:], pads], axis=1)

    agg1 = _sc_partial_segment_sum(features, real3, ex3)
    x1 = _tc_gin_mlp(features, agg1, W1a, b1a, W2a, b2a, eps1, relu_out=True)
    agg2 = _sc_partial_segment_sum(x1, real3, ex3)
    return _tc_gin_mlp(x1, agg2, W1b, b1b, W2b, b2b, eps2, relu_out=False)
